# Initial kernel scaffold; baseline (speedup 1.0000x reference)
import jax
import jax.numpy as jnp
from jax.experimental import pallas as pl


def kernel(x, edge_index, batch, W_proj, b_proj, ggc_W, gru_W_ih, gru_W_hh, gru_b_ih, gru_b_hh, attn_W1, attn_b1, attn_W2, attn_b2, cls_W1, cls_b1, cls_W2, cls_b2, cls_W3, cls_b3):
    raise NotImplementedError("write your pallas kernel here")


# speedup vs baseline: 5.4397x; 5.4397x over previous
import functools

import jax
import jax.numpy as jnp
from jax import lax
from jax.experimental import pallas as pl
from jax.experimental.pallas import tpu as pltpu
from jax.experimental.pallas import tpu_sc as plsc

N = 10000
E = 320000
D_IN = 128
H = 256
HH = H // 2
NG = 64
STEPS = 5

NUM_TILES = 16
CHUNK = 128
IB = 32
CH = 160
E_PAD = NUM_TILES * CHUNK * CH
NP = N + 112
ROWS_PER_TILE = NP // NUM_TILES

BLK = 1000
GRID = N // BLK

_f32 = jnp.float32


def _sc_body(ma_hbm, mb_hbm, src_hbm, dst_hbm, z_hbm, outa_hbm, outb_hbm,
             src_v, dst_v, rows_v, agg_sh, sem):
    c = lax.axis_index("c")
    s = lax.axis_index("s")
    row0 = s * ROWS_PER_TILE

    def run(m_hbm, out_hbm):
        pltpu.sync_copy(z_hbm.at[pl.ds(row0, ROWS_PER_TILE)],
                        agg_sh.at[pl.ds(row0, ROWS_PER_TILE)])
        plsc.subcore_barrier()

        def blk(b, carry):
            pltpu.sync_copy(src_hbm.at[s, pl.ds(b * IB, IB)], src_v)
            pltpu.sync_copy(dst_hbm.at[s, pl.ds(b * IB, IB)], dst_v)

            def body(j, carry2):
                pltpu.async_copy(m_hbm.at[src_v.at[j]], rows_v, sem).wait()
                pltpu.sync_copy(rows_v, agg_sh.at[dst_v.at[j]], add=True)
                return carry2

            lax.fori_loop(0, IB, body, 0)
            return carry

        lax.fori_loop(0, CH // IB, blk, 0)
        plsc.subcore_barrier()
        pltpu.sync_copy(agg_sh.at[pl.ds(row0, ROWS_PER_TILE)],
                        out_hbm.at[pl.ds(row0, ROWS_PER_TILE)])

    @pl.when(c == 0)
    def _():
        run(ma_hbm, outa_hbm)

    @pl.when(c == 1)
    def _():
        run(mb_hbm, outb_hbm)


@functools.cache
def _get_sc_segment_sum():
    return pl.kernel(
        _sc_body,
        out_type=[jax.ShapeDtypeStruct((NP, H // 2), _f32),
                  jax.ShapeDtypeStruct((NP, H // 2), _f32)],
        mesh=plsc.VectorSubcoreMesh(core_axis_name="c", subcore_axis_name="s"),
        scratch_types=[
            pltpu.VMEM((IB, CHUNK), jnp.int32),
            pltpu.VMEM((IB, CHUNK), jnp.int32),
            pltpu.VMEM((CHUNK, H // 2), _f32),
            pltpu.VMEM_SHARED((NP, H // 2), _f32),
            pltpu.SemaphoreType.DMA,
        ],
    )


def _sigmoid(x):
    return 1.0 / (1.0 + jnp.exp(-x))


def _dot(a, b):
    return jnp.dot(a, b, preferred_element_type=_f32)


def _t1_body(x_ref, wp_ref, bp_ref, w0_ref, h_ref, ma_ref, mb_ref):
    h = jnp.maximum(_dot(x_ref[...], wp_ref[...]) + bp_ref[...], 0.0)
    h_ref[...] = h
    m = _dot(h, w0_ref[...])
    ma_ref[...] = m[:, :H // 2]
    mb_ref[...] = m[:, H // 2:]


def _gru(aa, ab, h, wia, wib, whh, bih, bhh):
    gi = _dot(aa, wia) + _dot(ab, wib) + bih
    gh = _dot(h, whh) + bhh
    r = _sigmoid(gi[:, :H] + gh[:, :H])
    z = _sigmoid(gi[:, H:2 * H] + gh[:, H:2 * H])
    n = jnp.tanh(gi[:, 2 * H:] + r * gh[:, 2 * H:])
    return (1.0 - z) * n + z * h


def _t2a_body(aa_ref, ab_ref, h_ref, wia_ref, wib_ref, whh_ref, bih_ref,
              bhh_ref, wn_ref, ho_ref, ma_ref, mb_ref):
    hn = _gru(aa_ref[...], ab_ref[...], h_ref[...], wia_ref[...], wib_ref[...],
              whh_ref[...], bih_ref[...], bhh_ref[...])
    ho_ref[...] = hn
    m = _dot(hn, wn_ref[...])
    ma_ref[...] = m[:, :H // 2]
    mb_ref[...] = m[:, H // 2:]


def _t2b_body(aa_ref, ab_ref, h_ref, wia_ref, wib_ref, whh_ref, bih_ref,
              bhh_ref, aw1_ref, ab1_ref, aw2_ref, ab2_ref, bat_ref,
              awo_ref, sums_ref, maxs_ref, cnts_ref):
    i = pl.program_id(0)
    hn = _gru(aa_ref[...], ab_ref[...], h_ref[...], wia_ref[...], wib_ref[...],
              whh_ref[...], bih_ref[...], bhh_ref[...])
    hr = jnp.maximum(hn, 0.0)
    ah = jnp.maximum(_dot(hr, aw1_ref[...]) + ab1_ref[...], 0.0)
    aw = _sigmoid(_dot(ah, aw2_ref[...]) + ab2_ref[...])
    awo_ref[...] = aw
    wx = hr * aw

    bat = bat_ref[...]
    onehot = (bat == lax.broadcasted_iota(jnp.int32, (1, NG), 1)).astype(_f32)

    @pl.when(i == 0)
    def _():
        sums_ref[...] = jnp.zeros_like(sums_ref)
        maxs_ref[...] = jnp.full_like(maxs_ref, -jnp.inf)
        cnts_ref[...] = jnp.zeros_like(cnts_ref)

    sums_ref[...] += lax.dot_general(onehot, wx, (((0,), (0,)), ((), ())),
                                     preferred_element_type=_f32)
    cnts_ref[...] += jnp.sum(onehot, axis=0)[:, None]

    g_lo = bat_ref[0, 0]
    g_hi = bat_ref[BLK - 1, 0]

    def body(g, carry):
        v = jnp.max(jnp.where(bat == g, wx, -jnp.inf), axis=0, keepdims=True)
        rowmask = lax.broadcasted_iota(jnp.int32, (NG, 1), 0) == g
        cur = maxs_ref[...]
        maxs_ref[...] = jnp.where(rowmask, jnp.maximum(cur, v), cur)
        return carry

    lax.fori_loop(g_lo, g_hi + 1, body, 0)


def _t3_body(sums_ref, maxs_ref, cnts_ref, w1a_ref, w1b_ref, b1_ref, w2_ref,
             b2_ref, w3_ref, b3_ref, out_ref):
    c = cnts_ref[...]
    mean = sums_ref[...] / jnp.maximum(c, 1.0)
    mx = jnp.where(c > 0.0, maxs_ref[...], 0.0)
    g = jnp.maximum(_dot(mean, w1a_ref[...]) + _dot(mx, w1b_ref[...])
                    + b1_ref[...], 0.0)
    g = jnp.maximum(_dot(g, w2_ref[...]) + b2_ref[...], 0.0)
    out_ref[...] = _dot(g, w3_ref[...]) + b3_ref[...]


def _full(shape):
    return pl.BlockSpec(shape, lambda i: (0,) * len(shape))


def _rows(width):
    return pl.BlockSpec((BLK, width), lambda i: (i, 0))


_t1 = pl.pallas_call(
    _t1_body,
    grid=(GRID,),
    in_specs=[_rows(D_IN), _full((D_IN, H)), _full((1, H)), _full((H, H))],
    out_specs=[_rows(H), _rows(H // 2), _rows(H // 2)],
    out_shape=[jax.ShapeDtypeStruct((N, H), _f32),
               jax.ShapeDtypeStruct((N, H // 2), _f32),
               jax.ShapeDtypeStruct((N, H // 2), _f32)],
)

_t2a = pl.pallas_call(
    _t2a_body,
    grid=(GRID,),
    in_specs=[_rows(H // 2), _rows(H // 2), _rows(H),
              _full((H // 2, 3 * H)), _full((H // 2, 3 * H)),
              _full((H, 3 * H)), _full((1, 3 * H)), _full((1, 3 * H)),
              _full((H, H))],
    out_specs=[_rows(H), _rows(H // 2), _rows(H // 2)],
    out_shape=[jax.ShapeDtypeStruct((N, H), _f32),
               jax.ShapeDtypeStruct((N, H // 2), _f32),
               jax.ShapeDtypeStruct((N, H // 2), _f32)],
)

_t2b = pl.pallas_call(
    _t2b_body,
    grid=(GRID,),
    in_specs=[_rows(H // 2), _rows(H // 2), _rows(H),
              _full((H // 2, 3 * H)), _full((H // 2, 3 * H)),
              _full((H, 3 * H)), _full((1, 3 * H)), _full((1, 3 * H)),
              _full((H, HH)), _full((1, HH)), _full((HH, 1)), _full((1, 1)),
              _rows(1)],
    out_specs=[_rows(1), _full((NG, H)), _full((NG, H)), _full((NG, 1))],
    out_shape=[jax.ShapeDtypeStruct((N, 1), _f32),
               jax.ShapeDtypeStruct((NG, H), _f32),
               jax.ShapeDtypeStruct((NG, H), _f32),
               jax.ShapeDtypeStruct((NG, 1), _f32)],
)

_t3 = pl.pallas_call(
    _t3_body,
    grid=(1,),
    in_specs=[_full((NG, H)), _full((NG, H)), _full((NG, 1)),
              _full((H, H)), _full((H, H)), _full((1, H)),
              _full((H, HH)), _full((1, HH)), _full((HH, 2)), _full((1, 2))],
    out_specs=[_full((NG, 2))],
    out_shape=[jax.ShapeDtypeStruct((NG, 2), _f32)],
)


def kernel(x, edge_index, batch, W_proj, b_proj, ggc_W, gru_W_ih, gru_W_hh,
           gru_b_ih, gru_b_hh, attn_W1, attn_b1, attn_W2, attn_b2, cls_W1,
           cls_b1, cls_W2, cls_b2, cls_W3, cls_b3):
    src = edge_index[0]
    dst = edge_index[1]
    pad = E_PAD - E
    pad_ids = jnp.arange(pad, dtype=jnp.int32)
    src_t = jnp.concatenate([src, (pad_ids * 97) % N]).reshape(NUM_TILES, CH, CHUNK)
    dst_t = jnp.concatenate([dst, N + (pad_ids % (NP - N))]).reshape(NUM_TILES, CH, CHUNK)
    zeros_np = jnp.zeros((NP, H // 2), _f32)

    wihT = gru_W_ih.T
    wia, wib = wihT[:H // 2], wihT[H // 2:]
    whhT = gru_W_hh.T
    bih = gru_b_ih.reshape(1, 3 * H)
    bhh = gru_b_hh.reshape(1, 3 * H)

    sc_segment_sum = _get_sc_segment_sum()
    h, ma, mb = _t1(x, W_proj, b_proj.reshape(1, H), ggc_W[0])
    for i in range(STEPS):
        agg_a, agg_b = sc_segment_sum(ma, mb, src_t, dst_t, zeros_np)
        if i < STEPS - 1:
            h, ma, mb = _t2a(agg_a, agg_b, h, wia, wib, whhT, bih, bhh,
                             ggc_W[i + 1])
        else:
            aw, sums, maxs, cnts = _t2b(
                agg_a, agg_b, h, wia, wib, whhT, bih, bhh,
                attn_W1, attn_b1.reshape(1, HH), attn_W2,
                attn_b2.reshape(1, 1), batch.reshape(N, 1))

    preds = _t3(sums, maxs, cnts, cls_W1[:H], cls_W1[H:],
                cls_b1.reshape(1, H), cls_W2, cls_b2.reshape(1, HH),
                cls_W3, cls_b3.reshape(1, 2))[0]
    return preds, aw.reshape(N)
